# Initial kernel scaffold; baseline (speedup 1.0000x reference)
import jax
import jax.numpy as jnp
from jax.experimental import pallas as pl


def kernel(x_ns, weight):
    raise NotImplementedError("write your pallas kernel here")


# speedup vs baseline: 3.9259x; 3.9259x over previous
import functools

import jax
import jax.numpy as jnp
from jax import lax
from jax.experimental import pallas as pl
from jax.experimental.pallas import tpu as pltpu
from jax.experimental.pallas import tpu_sc as plsc

_INFO = plsc.get_sparse_core_info()
_NC = _INFO.num_cores
_NS = _INFO.num_subcores
_NW = _NC * _NS

_B = 4096 * 200
_D = 128
_GRP = 128
_ROWS = _B // _D
_GPW = _ROWS // _NW


def _body(x_hbm, tbl_hbm, out_hbm, idx_v, rows_v, sem):
    wid = lax.axis_index("s") * _NC + lax.axis_index("c")
    base = wid * _GPW
    pltpu.sync_copy(x_hbm.at[pl.ds(base, _GPW)], idx_v)

    def grp(j, carry):
        pltpu.async_copy(tbl_hbm.at[idx_v.at[j]], rows_v, sem).wait()
        pltpu.sync_copy(rows_v, out_hbm.at[pl.ds((base + j) * _GRP, _GRP)])
        return carry

    lax.fori_loop(0, _GPW, grp, 0)


@jax.jit
def _lookup(x2d, weight):
    k = pl.kernel(
        _body,
        out_type=jax.ShapeDtypeStruct((_B, _D), jnp.float32),
        mesh=plsc.VectorSubcoreMesh(core_axis_name="c", subcore_axis_name="s"),
        scratch_types=[
            pltpu.VMEM((_GPW, _GRP), jnp.int32),
            pltpu.VMEM((_GRP, _D), jnp.float32),
            pltpu.SemaphoreType.DMA,
        ],
    )
    return k(x2d, weight)


def kernel(x_ns, weight):
    n, s = x_ns.shape
    x2d = x_ns.astype(jnp.int32).reshape(_ROWS, _GRP)
    out = _lookup(x2d, weight)
    return out.reshape(n, s, _D)


# speedup vs baseline: 4.1162x; 1.0485x over previous
import functools

import jax
import jax.numpy as jnp
from jax import lax
from jax.experimental import pallas as pl
from jax.experimental.pallas import tpu as pltpu
from jax.experimental.pallas import tpu_sc as plsc

_INFO = plsc.get_sparse_core_info()
_NC = _INFO.num_cores
_NS = _INFO.num_subcores
_NW = _NC * _NS

_B = 4096 * 200
_D = 128
_GRP = 128
_ROWS = _B // _D
_GPW = _ROWS // _NW
_NBUF = 4
_NOUT = _GPW // _NBUF


def _body(x_hbm, tbl_hbm, out_hbm, idx_v, bufs, gsem, wsem):
    wid = lax.axis_index("s") * _NC + lax.axis_index("c")
    base = wid * _GPW
    pltpu.sync_copy(x_hbm.at[pl.ds(base, _GPW)], idx_v)

    for b in range(_NBUF):
        pltpu.async_copy(tbl_hbm.at[idx_v.at[b]], bufs.at[b], gsem.at[b])

    def outer(g, carry):
        for b in range(_NBUF):
            j = g * _NBUF + b
            pltpu.make_async_copy(
                tbl_hbm.at[idx_v.at[b]], bufs.at[b], gsem.at[b]
            ).wait()
            dst = out_hbm.at[pl.ds((base + j) * _GRP, _GRP)]
            pltpu.async_copy(bufs.at[b], dst, wsem.at[b])

            @pl.when(g + 1 < _NOUT)
            def _():
                pltpu.make_async_copy(bufs.at[b], dst, wsem.at[b]).wait()
                pltpu.async_copy(
                    tbl_hbm.at[idx_v.at[j + _NBUF]], bufs.at[b], gsem.at[b]
                )

        return carry

    lax.fori_loop(0, _NOUT, outer, 0)

    for b in range(_NBUF):
        j = _GPW - _NBUF + b
        pltpu.make_async_copy(
            bufs.at[b],
            out_hbm.at[pl.ds((base + j) * _GRP, _GRP)],
            wsem.at[b],
        ).wait()


@jax.jit
def _lookup(x2d, weight):
    k = pl.kernel(
        _body,
        out_type=jax.ShapeDtypeStruct((_B, _D), jnp.float32),
        mesh=plsc.VectorSubcoreMesh(core_axis_name="c", subcore_axis_name="s"),
        scratch_types=[
            pltpu.VMEM((_GPW, _GRP), jnp.int32),
            pltpu.VMEM((_NBUF, _GRP, _D), jnp.float32),
            pltpu.SemaphoreType.DMA((_NBUF,)),
            pltpu.SemaphoreType.DMA((_NBUF,)),
        ],
    )
    return k(x2d, weight)


def kernel(x_ns, weight):
    n, s = x_ns.shape
    x2d = x_ns.astype(jnp.int32).reshape(_ROWS, _GRP)
    out = _lookup(x2d, weight)
    return out.reshape(n, s, _D)


# speedup vs baseline: 16.0944x; 3.9100x over previous
import functools

import jax
import jax.numpy as jnp
from jax import lax
from jax.experimental import pallas as pl
from jax.experimental.pallas import tpu as pltpu
from jax.experimental.pallas import tpu_sc as plsc

_INFO = plsc.get_sparse_core_info()
_NC = _INFO.num_cores
_NS = _INFO.num_subcores
_NW = _NC * _NS

_B = 4096 * 200
_D = 128
_GRP = 128
_ROWS = _B // _D
_GPW = _ROWS // _NW
_NBUF = 4
_NOUT = _GPW // _NBUF


def _body(x_hbm, tbl_hbm, out_hbm, idx_v, bufs, tbl_sh, gsem, wsem):
    sid = lax.axis_index("s")
    wid = sid * _NC + lax.axis_index("c")
    base = wid * _GPW

    @pl.when(sid == 0)
    def _():
        pltpu.sync_copy(tbl_hbm, tbl_sh)

    pltpu.sync_copy(x_hbm.at[pl.ds(base, _GPW)], idx_v)
    plsc.subcore_barrier()

    for b in range(_NBUF):
        pltpu.async_copy(tbl_sh.at[idx_v.at[b]], bufs.at[b], gsem.at[b])

    def outer(g, carry):
        for b in range(_NBUF):
            j = g * _NBUF + b
            pltpu.make_async_copy(
                tbl_sh.at[idx_v.at[b]], bufs.at[b], gsem.at[b]
            ).wait()
            dst = out_hbm.at[pl.ds((base + j) * _GRP, _GRP)]
            pltpu.async_copy(bufs.at[b], dst, wsem.at[b])

            @pl.when(g + 1 < _NOUT)
            def _():
                pltpu.make_async_copy(bufs.at[b], dst, wsem.at[b]).wait()
                pltpu.async_copy(
                    tbl_sh.at[idx_v.at[j + _NBUF]], bufs.at[b], gsem.at[b]
                )

        return carry

    lax.fori_loop(0, _NOUT, outer, 0)

    for b in range(_NBUF):
        j = _GPW - _NBUF + b
        pltpu.make_async_copy(
            bufs.at[b],
            out_hbm.at[pl.ds((base + j) * _GRP, _GRP)],
            wsem.at[b],
        ).wait()


@jax.jit
def _lookup(x2d, weight):
    k = pl.kernel(
        _body,
        out_type=jax.ShapeDtypeStruct((_B, _D), jnp.float32),
        mesh=plsc.VectorSubcoreMesh(core_axis_name="c", subcore_axis_name="s"),
        scratch_types=[
            pltpu.VMEM((_GPW, _GRP), jnp.int32),
            pltpu.VMEM((_NBUF, _GRP, _D), jnp.float32),
            pltpu.VMEM_SHARED((255, _D), jnp.float32),
            pltpu.SemaphoreType.DMA((_NBUF,)),
            pltpu.SemaphoreType.DMA((_NBUF,)),
        ],
    )
    return k(x2d, weight)


def kernel(x_ns, weight):
    n, s = x_ns.shape
    x2d = x_ns.astype(jnp.int32).reshape(_ROWS, _GRP)
    out = _lookup(x2d, weight)
    return out.reshape(n, s, _D)
